# baseline (speedup 1.0000x reference)
import numpy as np
import jax
import jax.numpy as jnp
from jax import lax
from jax.experimental import pallas as pl
from jax.experimental.pallas import tpu as pltpu
from jax.experimental.pallas import tpu_sc as plsc

B, C, N = 4, 3, 4096
L = 16
NC, NS = 2, 16
NW = NC * NS
WPB = NW // B
QPW = N // WPB
G = 4
NJG = N // L
BIG = 3.0e38


def _bf16_round_f32(x):
    u = jax.lax.bitcast_convert_type(x, jnp.uint32)
    r = u + np.uint32(0x7FFF) + ((u >> np.uint32(16)) & np.uint32(1))
    r = r & np.uint32(0xFFFF0000)
    return jax.lax.bitcast_convert_type(r, jnp.float32)


def _sc_body(in_hbm, inb_hbm, tgt_hbm, tgtb_hbm, out_hbm,
             qx, qy, qz, qbx, qby, qbz, tx, ty, tz, tbx, tby, tbz, tn,
             colmin, red, treebuf, outv, shared):
    c = lax.axis_index("c")
    s = lax.axis_index("s")
    batch = NC * c + s // WPB
    row0 = (s % WPB) * QPW
    qoff = batch * (C * N) + row0
    toff = batch * (C * N)

    pltpu.sync_copy(in_hbm.at[pl.ds(qoff, QPW)], qx)
    pltpu.sync_copy(in_hbm.at[pl.ds(qoff + N, QPW)], qy)
    pltpu.sync_copy(in_hbm.at[pl.ds(qoff + 2 * N, QPW)], qz)
    pltpu.sync_copy(inb_hbm.at[pl.ds(qoff, QPW)], qbx)
    pltpu.sync_copy(inb_hbm.at[pl.ds(qoff + N, QPW)], qby)
    pltpu.sync_copy(inb_hbm.at[pl.ds(qoff + 2 * N, QPW)], qbz)
    pltpu.sync_copy(tgt_hbm.at[pl.ds(toff, N)], tx)
    pltpu.sync_copy(tgt_hbm.at[pl.ds(toff + N, N)], ty)
    pltpu.sync_copy(tgt_hbm.at[pl.ds(toff + 2 * N, N)], tz)
    pltpu.sync_copy(tgtb_hbm.at[pl.ds(toff, N)], tbx)
    pltpu.sync_copy(tgtb_hbm.at[pl.ds(toff + N, N)], tby)
    pltpu.sync_copy(tgtb_hbm.at[pl.ds(toff + 2 * N, N)], tbz)

    def prep(g, carry):
        o = g * L
        vx = tx[pl.ds(o, L)]
        vy = ty[pl.ds(o, L)]
        vz = tz[pl.ds(o, L)]
        tn[pl.ds(o, L)] = vx * vx + vy * vy + vz * vz
        colmin[pl.ds(o, L)] = jnp.full((L,), BIG, jnp.float32)
        return carry
    lax.fori_loop(0, NJG, prep, 0)
    for k in range(G):
        treebuf[pl.ds(k * 2 * L + L, L)] = jnp.full((L,), BIG, jnp.float32)
    lane = lax.iota(jnp.int32, L)

    def qblock(qb, rowsum):
        base = qb * L
        qvx = qx[pl.ds(base, L)]
        qvy = qy[pl.ds(base, L)]
        qvz = qz[pl.ds(base, L)]
        qvbx = qbx[pl.ds(base, L)]
        qvby = qby[pl.ds(base, L)]
        qvbz = qbz[pl.ds(base, L)]
        for sub in range(L // G):
            bx, by, bz, qn = [], [], [], []
            for k in range(G):
                j = sub * G + k
                gx, gy, gz = qvx[j], qvy[j], qvz[j]
                qn.append(jnp.full((L,), gx * gx + gy * gy + gz * gz,
                                   jnp.float32))
                bx.append(jnp.full((L,), qvbx[j], jnp.float32))
                by.append(jnp.full((L,), qvby[j], jnp.float32))
                bz.append(jnp.full((L,), qvbz[j], jnp.float32))

            def jsweep(jg, rmins):
                o = jg * L
                vtx = tbx[pl.ds(o, L)]
                vty = tby[pl.ds(o, L)]
                vtz = tbz[pl.ds(o, L)]
                vtn = tn[pl.ds(o, L)]
                cm = colmin[pl.ds(o, L)]
                out = []
                for k in range(G):
                    m = bx[k] * vtx + by[k] * vty + bz[k] * vtz
                    d = (qn[k] + vtn) - 2.0 * m
                    out.append(jnp.minimum(rmins[k], d))
                    cm = jnp.minimum(cm, d)
                colmin[pl.ds(o, L)] = cm
                return tuple(out)

            rmins = lax.fori_loop(
                0, NJG, jsweep, tuple(jnp.full((L,), BIG, jnp.float32)
                                      for _ in range(G)))
            for k in range(G):
                treebuf[pl.ds(k * 2 * L, L)] = rmins[k]
            for sh in (8, 4, 2, 1):
                vs = [jnp.minimum(treebuf[pl.ds(k * 2 * L, L)],
                                  treebuf[pl.ds(k * 2 * L + sh, L)])
                      for k in range(G)]
                for k in range(G):
                    treebuf[pl.ds(k * 2 * L, L)] = vs[k]
            for k in range(G):
                v = treebuf[pl.ds(k * 2 * L, L)]
                rowsum = rowsum + jnp.where(lane == 0, v, 0.0)
        return rowsum

    rowsum = lax.fori_loop(0, QPW // L, qblock,
                           jnp.zeros((L,), jnp.float32))

    pltpu.sync_copy(colmin, shared.at[pl.ds(s * N, N)])
    plsc.subcore_barrier()
    rbase = (s // WPB) * WPB
    j0 = (s % WPB) * QPW
    for r in range(WPB):
        pltpu.sync_copy(shared.at[pl.ds((rbase + r) * N + j0, QPW)],
                        red.at[pl.ds(r * QPW, QPW)])

    def colred(g, csum):
        o = g * L
        m = red[pl.ds(o, L)]
        for r in range(1, WPB):
            m = jnp.minimum(m, red[pl.ds(r * QPW + o, L)])
        return csum + m
    csum = lax.fori_loop(0, QPW // L, colred,
                         jnp.zeros((L,), jnp.float32))

    outv[...] = csum + rowsum
    w = c * NS + s
    pltpu.sync_copy(outv, out_hbm.at[pl.ds(w * L, L)])


def kernel(in_pc, target_pc):
    mesh = plsc.VectorSubcoreMesh(core_axis_name="c", subcore_axis_name="s")
    fn = pl.kernel(
        _sc_body, mesh=mesh,
        out_type=jax.ShapeDtypeStruct((NW * L,), jnp.float32),
        scratch_types=[
            pltpu.VMEM((QPW,), jnp.float32),
            pltpu.VMEM((QPW,), jnp.float32),
            pltpu.VMEM((QPW,), jnp.float32),
            pltpu.VMEM((QPW,), jnp.float32),
            pltpu.VMEM((QPW,), jnp.float32),
            pltpu.VMEM((QPW,), jnp.float32),
            pltpu.VMEM((N,), jnp.float32),
            pltpu.VMEM((N,), jnp.float32),
            pltpu.VMEM((N,), jnp.float32),
            pltpu.VMEM((N,), jnp.float32),
            pltpu.VMEM((N,), jnp.float32),
            pltpu.VMEM((N,), jnp.float32),
            pltpu.VMEM((N,), jnp.float32),
            pltpu.VMEM((N,), jnp.float32),
            pltpu.VMEM((WPB * QPW,), jnp.float32),
            pltpu.VMEM((G * 2 * L,), jnp.float32),
            pltpu.VMEM((L,), jnp.float32),
            pltpu.VMEM_SHARED((NS * N,), jnp.float32),
        ],
    )
    in_flat = in_pc.reshape(-1)
    tgt_flat = target_pc.reshape(-1)
    in_rnd = _bf16_round_f32(in_flat)
    tgt_rnd = _bf16_round_f32(tgt_flat)
    parts = fn(in_flat, in_rnd, tgt_flat, tgt_rnd)
    return jnp.sum(parts) / (2.0 * B * N)


# speedup vs baseline: 2.8387x; 2.8387x over previous
import numpy as np
import jax
import jax.numpy as jnp
from jax import lax
from jax.experimental import pallas as pl
from jax.experimental.pallas import tpu as pltpu
from jax.experimental.pallas import tpu_sc as plsc

B, C, N = 4, 3, 4096
TILE = 1024
TCB = 3
L = 16
NC, NS = 2, 16
NW = NC * NS
QPW = N // NW
G = 4
NJG = N // L
SPW = N // NS
BIG = 3.0e38


def _tc_body(p1t_ref, p2_ref, loss_ref, colmin_ref):
    b = pl.program_id(0)
    i = pl.program_id(1)
    nt = pl.num_programs(1)

    p = p1t_ref[0]
    q = p2_ref[0]

    pb = p.astype(jnp.bfloat16)
    qb = q.astype(jnp.bfloat16)
    cross = jax.lax.dot_general(
        pb, qb, (((1,), (0,)), ((), ())),
        preferred_element_type=jnp.float32)
    pn = (p[:, 0:1] * p[:, 0:1]
          + p[:, 1:2] * p[:, 1:2]
          + p[:, 2:3] * p[:, 2:3])
    qn = (q[0:1, :] * q[0:1, :]
          + q[1:2, :] * q[1:2, :]
          + q[2:3, :] * q[2:3, :])
    d = ((-2.0) * cross + pn) + qn

    row_min = jnp.min(d, axis=1)
    col_min = jnp.min(d, axis=0)

    prev_col = colmin_ref[0, 0, :]
    new_col = jnp.where(i == 0, col_min, jnp.minimum(prev_col, col_min))
    colmin_ref[0, 0, :] = new_col

    acc = jnp.where((b == 0) & (i == 0), 0.0, loss_ref[0, 0])
    acc = acc + jnp.sum(row_min)
    acc = acc + jnp.where(i == nt - 1, jnp.sum(new_col), 0.0)
    loss_ref[0, 0] = acc


def _tc_part(in_pc, target_pc):
    nb = in_pc.shape[0]
    nt = N // TILE
    pc1_t = jnp.transpose(in_pc, (0, 2, 1))
    loss, _ = pl.pallas_call(
        _tc_body,
        grid=(nb, nt),
        in_specs=[
            pl.BlockSpec((1, TILE, C), lambda b, i: (b, i, 0)),
            pl.BlockSpec((1, C, N), lambda b, i: (b, 0, 0)),
        ],
        out_specs=[
            pl.BlockSpec((1, 1), lambda b, i: (0, 0),
                         memory_space=pltpu.SMEM),
            pl.BlockSpec((1, 1, N), lambda b, i: (b, 0, 0)),
        ],
        out_shape=[
            jax.ShapeDtypeStruct((1, 1), jnp.float32),
            jax.ShapeDtypeStruct((nb, 1, N), jnp.float32),
        ],
    )(pc1_t, target_pc)
    return loss[0, 0]


def _bf16_round_f32(x):
    u = jax.lax.bitcast_convert_type(x, jnp.uint32)
    r = u + np.uint32(0x7FFF) + ((u >> np.uint32(16)) & np.uint32(1))
    r = r & np.uint32(0xFFFF0000)
    return jax.lax.bitcast_convert_type(r, jnp.float32)


def _sc_body(in_hbm, inb_hbm, tgt_hbm, tgtb_hbm, out_hbm,
             qx, qy, qz, qbx, qby, qbz, tx, ty, tz, tbx, tby, tbz, tn,
             colmin, cmfin, treebuf, outv, shared):
    c = lax.axis_index("c")
    s = lax.axis_index("s")
    w = c * NS + s
    row0 = w * QPW
    qoff = row0

    pltpu.sync_copy(in_hbm.at[pl.ds(qoff, QPW)], qx)
    pltpu.sync_copy(in_hbm.at[pl.ds(qoff + N, QPW)], qy)
    pltpu.sync_copy(in_hbm.at[pl.ds(qoff + 2 * N, QPW)], qz)
    pltpu.sync_copy(inb_hbm.at[pl.ds(qoff, QPW)], qbx)
    pltpu.sync_copy(inb_hbm.at[pl.ds(qoff + N, QPW)], qby)
    pltpu.sync_copy(inb_hbm.at[pl.ds(qoff + 2 * N, QPW)], qbz)
    pltpu.sync_copy(tgt_hbm.at[pl.ds(0, N)], tx)
    pltpu.sync_copy(tgt_hbm.at[pl.ds(N, N)], ty)
    pltpu.sync_copy(tgt_hbm.at[pl.ds(2 * N, N)], tz)
    pltpu.sync_copy(tgtb_hbm.at[pl.ds(0, N)], tbx)
    pltpu.sync_copy(tgtb_hbm.at[pl.ds(N, N)], tby)
    pltpu.sync_copy(tgtb_hbm.at[pl.ds(2 * N, N)], tbz)

    def prep(g, carry):
        o = g * L
        vx = tx[pl.ds(o, L)]
        vy = ty[pl.ds(o, L)]
        vz = tz[pl.ds(o, L)]
        tn[pl.ds(o, L)] = vx * vx + vy * vy + vz * vz
        colmin[pl.ds(o, L)] = jnp.full((L,), BIG, jnp.float32)
        return carry
    lax.fori_loop(0, NJG, prep, 0)
    for k in range(G):
        treebuf[pl.ds(k * 2 * L + L, L)] = jnp.full((L,), BIG, jnp.float32)
    lane = lax.iota(jnp.int32, L)

    def qblock(qb, rowsum):
        base = qb * L
        qvx = qx[pl.ds(base, L)]
        qvy = qy[pl.ds(base, L)]
        qvz = qz[pl.ds(base, L)]
        qvbx = qbx[pl.ds(base, L)]
        qvby = qby[pl.ds(base, L)]
        qvbz = qbz[pl.ds(base, L)]
        for sub in range(L // G):
            bx, by, bz, qn = [], [], [], []
            for k in range(G):
                j = sub * G + k
                gx, gy, gz = qvx[j], qvy[j], qvz[j]
                qn.append(jnp.full((L,), gx * gx + gy * gy + gz * gz,
                                   jnp.float32))
                bx.append(jnp.full((L,), qvbx[j], jnp.float32))
                by.append(jnp.full((L,), qvby[j], jnp.float32))
                bz.append(jnp.full((L,), qvbz[j], jnp.float32))

            def jsweep(jg, rmins):
                o = jg * L
                vtx = tbx[pl.ds(o, L)]
                vty = tby[pl.ds(o, L)]
                vtz = tbz[pl.ds(o, L)]
                vtn = tn[pl.ds(o, L)]
                cm = colmin[pl.ds(o, L)]
                out = []
                for k in range(G):
                    m = bx[k] * vtx + by[k] * vty + bz[k] * vtz
                    d = (qn[k] + vtn) - 2.0 * m
                    out.append(jnp.minimum(rmins[k], d))
                    cm = jnp.minimum(cm, d)
                colmin[pl.ds(o, L)] = cm
                return tuple(out)

            rmins = lax.fori_loop(
                0, NJG, jsweep, tuple(jnp.full((L,), BIG, jnp.float32)
                                      for _ in range(G)))
            for k in range(G):
                treebuf[pl.ds(k * 2 * L, L)] = rmins[k]
            for sh in (8, 4, 2, 1):
                vs = [jnp.minimum(treebuf[pl.ds(k * 2 * L, L)],
                                  treebuf[pl.ds(k * 2 * L + sh, L)])
                      for k in range(G)]
                for k in range(G):
                    treebuf[pl.ds(k * 2 * L, L)] = vs[k]
            for k in range(G):
                v = treebuf[pl.ds(k * 2 * L, L)]
                rowsum = rowsum + jnp.where(lane == 0, v, 0.0)
        return rowsum

    rowsum = lax.fori_loop(0, QPW // L, qblock,
                           jnp.zeros((L,), jnp.float32))

    pltpu.sync_copy(colmin, shared.at[pl.ds(s * N, N)])
    plsc.subcore_barrier()
    j0 = s * SPW
    for r in range(NS):
        pltpu.sync_copy(shared.at[pl.ds(r * N + j0, SPW)],
                        cmfin.at[pl.ds(r * SPW, SPW)])

    def colred(g, carry):
        o = g * L
        m = cmfin[pl.ds(o, L)]
        for r in range(1, NS):
            m = jnp.minimum(m, cmfin[pl.ds(r * SPW + o, L)])
        cmfin[pl.ds(o, L)] = m
        return carry
    lax.fori_loop(0, SPW // L, colred, 0)

    pltpu.sync_copy(cmfin.at[pl.ds(0, SPW)],
                    out_hbm.at[pl.ds(c * N + j0, SPW)])
    outv[...] = rowsum
    pltpu.sync_copy(outv, out_hbm.at[pl.ds(NC * N + w * L, L)])


def _sc_part(in_b, inb_b, tgt_b, tgtb_b):
    mesh = plsc.VectorSubcoreMesh(core_axis_name="c", subcore_axis_name="s")
    fn = pl.kernel(
        _sc_body, mesh=mesh,
        out_type=jax.ShapeDtypeStruct((NC * N + NW * L,), jnp.float32),
        scratch_types=[
            pltpu.VMEM((QPW,), jnp.float32),
            pltpu.VMEM((QPW,), jnp.float32),
            pltpu.VMEM((QPW,), jnp.float32),
            pltpu.VMEM((QPW,), jnp.float32),
            pltpu.VMEM((QPW,), jnp.float32),
            pltpu.VMEM((QPW,), jnp.float32),
            pltpu.VMEM((N,), jnp.float32),
            pltpu.VMEM((N,), jnp.float32),
            pltpu.VMEM((N,), jnp.float32),
            pltpu.VMEM((N,), jnp.float32),
            pltpu.VMEM((N,), jnp.float32),
            pltpu.VMEM((N,), jnp.float32),
            pltpu.VMEM((N,), jnp.float32),
            pltpu.VMEM((N,), jnp.float32),
            pltpu.VMEM((NS * SPW,), jnp.float32),
            pltpu.VMEM((G * 2 * L,), jnp.float32),
            pltpu.VMEM((L,), jnp.float32),
            pltpu.VMEM_SHARED((NS * N,), jnp.float32),
        ],
    )
    return fn(in_b, inb_b, tgt_b, tgtb_b)


def kernel(in_pc, target_pc):
    tc_loss = _tc_part(in_pc[:TCB], target_pc[:TCB])

    in_b = in_pc[TCB].reshape(-1)
    tgt_b = target_pc[TCB].reshape(-1)
    inb_b = _bf16_round_f32(in_b)
    tgtb_b = _bf16_round_f32(tgt_b)
    sc_out = _sc_part(in_b, inb_b, tgt_b, tgtb_b)

    cm = jnp.minimum(sc_out[:N], sc_out[N:2 * N])
    sc_loss = jnp.sum(cm) + jnp.sum(sc_out[2 * N:])
    return (tc_loss + sc_loss) / (2.0 * B * N)


# speedup vs baseline: 2.9317x; 1.0328x over previous
import numpy as np
import jax
import jax.numpy as jnp
from jax import lax
from jax.experimental import pallas as pl
from jax.experimental.pallas import tpu as pltpu
from jax.experimental.pallas import tpu_sc as plsc

B, C, N = 4, 3, 4096
L = 16
NC, NS = 2, 16
NW = NC * NS
WPB = NW // B
SCR = 768
QPW = SCR // WPB
TCR = N - SCR
TILE = 256
G = 4
NJG = N // L
SPW = N // WPB
BIG = 3.0e38


def _tc_body(p1t_ref, p2_ref, loss_ref, colmin_ref):
    b = pl.program_id(0)
    i = pl.program_id(1)

    p = p1t_ref[0]
    q = p2_ref[0]

    pb = p.astype(jnp.bfloat16)
    qb = q.astype(jnp.bfloat16)
    cross = jax.lax.dot_general(
        pb, qb, (((1,), (0,)), ((), ())),
        preferred_element_type=jnp.float32)
    pn = (p[:, 0:1] * p[:, 0:1]
          + p[:, 1:2] * p[:, 1:2]
          + p[:, 2:3] * p[:, 2:3])
    qn = (q[0:1, :] * q[0:1, :]
          + q[1:2, :] * q[1:2, :]
          + q[2:3, :] * q[2:3, :])
    d = ((-2.0) * cross + pn) + qn

    row_min = jnp.min(d, axis=1)
    col_min = jnp.min(d, axis=0)

    prev_col = colmin_ref[0, 0, :]
    new_col = jnp.where(i == 0, col_min, jnp.minimum(prev_col, col_min))
    colmin_ref[0, 0, :] = new_col

    acc = jnp.where((b == 0) & (i == 0), 0.0, loss_ref[0, 0])
    loss_ref[0, 0] = acc + jnp.sum(row_min)


def _tc_part(p1t, target_pc):
    nt = TCR // TILE
    loss, colmin = pl.pallas_call(
        _tc_body,
        grid=(B, nt),
        in_specs=[
            pl.BlockSpec((1, TILE, C), lambda b, i: (b, i, 0)),
            pl.BlockSpec((1, C, N), lambda b, i: (b, 0, 0)),
        ],
        out_specs=[
            pl.BlockSpec((1, 1), lambda b, i: (0, 0),
                         memory_space=pltpu.SMEM),
            pl.BlockSpec((1, 1, N), lambda b, i: (b, 0, 0)),
        ],
        out_shape=[
            jax.ShapeDtypeStruct((1, 1), jnp.float32),
            jax.ShapeDtypeStruct((B, 1, N), jnp.float32),
        ],
    )(p1t, target_pc)
    return loss, colmin


def _bf16_round_f32(x):
    u = jax.lax.bitcast_convert_type(x, jnp.uint32)
    r = u + np.uint32(0x7FFF) + ((u >> np.uint32(16)) & np.uint32(1))
    r = r & np.uint32(0xFFFF0000)
    return jax.lax.bitcast_convert_type(r, jnp.float32)


def _sc_body(in_hbm, inb_hbm, tgt_hbm, tgtb_hbm, out_hbm,
             qx, qy, qz, qbx, qby, qbz, tx, ty, tz, tbx, tby, tbz, tn,
             colmin, red, treebuf, outv, shared):
    c = lax.axis_index("c")
    s = lax.axis_index("s")
    batch = NC * c + s // WPB
    qoff = batch * (C * N) + TCR + (s % WPB) * QPW
    toff = batch * (C * N)

    pltpu.sync_copy(in_hbm.at[pl.ds(qoff, QPW)], qx)
    pltpu.sync_copy(in_hbm.at[pl.ds(qoff + N, QPW)], qy)
    pltpu.sync_copy(in_hbm.at[pl.ds(qoff + 2 * N, QPW)], qz)
    pltpu.sync_copy(inb_hbm.at[pl.ds(qoff, QPW)], qbx)
    pltpu.sync_copy(inb_hbm.at[pl.ds(qoff + N, QPW)], qby)
    pltpu.sync_copy(inb_hbm.at[pl.ds(qoff + 2 * N, QPW)], qbz)
    pltpu.sync_copy(tgt_hbm.at[pl.ds(toff, N)], tx)
    pltpu.sync_copy(tgt_hbm.at[pl.ds(toff + N, N)], ty)
    pltpu.sync_copy(tgt_hbm.at[pl.ds(toff + 2 * N, N)], tz)
    pltpu.sync_copy(tgtb_hbm.at[pl.ds(toff, N)], tbx)
    pltpu.sync_copy(tgtb_hbm.at[pl.ds(toff + N, N)], tby)
    pltpu.sync_copy(tgtb_hbm.at[pl.ds(toff + 2 * N, N)], tbz)

    def prep(g, carry):
        o = g * L
        vx = tx[pl.ds(o, L)]
        vy = ty[pl.ds(o, L)]
        vz = tz[pl.ds(o, L)]
        tn[pl.ds(o, L)] = vx * vx + vy * vy + vz * vz
        colmin[pl.ds(o, L)] = jnp.full((L,), BIG, jnp.float32)
        return carry
    lax.fori_loop(0, NJG, prep, 0)
    for k in range(G):
        treebuf[pl.ds(k * 2 * L + L, L)] = jnp.full((L,), BIG, jnp.float32)
    lane = lax.iota(jnp.int32, L)

    def qblock(qb, rowsum):
        base = qb * L
        qvx = qx[pl.ds(base, L)]
        qvy = qy[pl.ds(base, L)]
        qvz = qz[pl.ds(base, L)]
        qvbx = qbx[pl.ds(base, L)]
        qvby = qby[pl.ds(base, L)]
        qvbz = qbz[pl.ds(base, L)]
        for sub in range(L // G):
            bx, by, bz, qn = [], [], [], []
            for k in range(G):
                j = sub * G + k
                gx, gy, gz = qvx[j], qvy[j], qvz[j]
                qn.append(jnp.full((L,), gx * gx + gy * gy + gz * gz,
                                   jnp.float32))
                bx.append(jnp.full((L,), qvbx[j], jnp.float32))
                by.append(jnp.full((L,), qvby[j], jnp.float32))
                bz.append(jnp.full((L,), qvbz[j], jnp.float32))

            def jsweep(jg, rmins):
                o = jg * L
                vtx = tbx[pl.ds(o, L)]
                vty = tby[pl.ds(o, L)]
                vtz = tbz[pl.ds(o, L)]
                vtn = tn[pl.ds(o, L)]
                cm = colmin[pl.ds(o, L)]
                out = []
                for k in range(G):
                    m = bx[k] * vtx + by[k] * vty + bz[k] * vtz
                    d = (qn[k] + vtn) - 2.0 * m
                    out.append(jnp.minimum(rmins[k], d))
                    cm = jnp.minimum(cm, d)
                colmin[pl.ds(o, L)] = cm
                return tuple(out)

            rmins = lax.fori_loop(
                0, NJG, jsweep, tuple(jnp.full((L,), BIG, jnp.float32)
                                      for _ in range(G)))
            for k in range(G):
                treebuf[pl.ds(k * 2 * L, L)] = rmins[k]
            for sh in (8, 4, 2, 1):
                vs = [jnp.minimum(treebuf[pl.ds(k * 2 * L, L)],
                                  treebuf[pl.ds(k * 2 * L + sh, L)])
                      for k in range(G)]
                for k in range(G):
                    treebuf[pl.ds(k * 2 * L, L)] = vs[k]
            for k in range(G):
                v = treebuf[pl.ds(k * 2 * L, L)]
                rowsum = rowsum + jnp.where(lane == 0, v, 0.0)
        return rowsum

    rowsum = lax.fori_loop(0, QPW // L, qblock,
                           jnp.zeros((L,), jnp.float32))

    pltpu.sync_copy(colmin, shared.at[pl.ds(s * N, N)])
    plsc.subcore_barrier()
    rbase = (s // WPB) * WPB
    j0 = (s % WPB) * SPW
    for r in range(WPB):
        pltpu.sync_copy(shared.at[pl.ds((rbase + r) * N + j0, SPW)],
                        red.at[pl.ds(r * SPW, SPW)])

    def colred(g, carry):
        o = g * L
        m = red[pl.ds(o, L)]
        for r in range(1, WPB):
            m = jnp.minimum(m, red[pl.ds(r * SPW + o, L)])
        red[pl.ds(o, L)] = m
        return carry
    lax.fori_loop(0, SPW // L, colred, 0)

    pltpu.sync_copy(red.at[pl.ds(0, SPW)],
                    out_hbm.at[pl.ds(batch * N + j0, SPW)])
    outv[...] = rowsum
    w = c * NS + s
    pltpu.sync_copy(outv, out_hbm.at[pl.ds(B * N + w * L, L)])


def _sc_part(in_flat, inb_flat, tgt_flat, tgtb_flat):
    mesh = plsc.VectorSubcoreMesh(core_axis_name="c", subcore_axis_name="s")
    fn = pl.kernel(
        _sc_body, mesh=mesh,
        out_type=jax.ShapeDtypeStruct((B * N + NW * L,), jnp.float32),
        scratch_types=[
            pltpu.VMEM((QPW,), jnp.float32),
            pltpu.VMEM((QPW,), jnp.float32),
            pltpu.VMEM((QPW,), jnp.float32),
            pltpu.VMEM((QPW,), jnp.float32),
            pltpu.VMEM((QPW,), jnp.float32),
            pltpu.VMEM((QPW,), jnp.float32),
            pltpu.VMEM((N,), jnp.float32),
            pltpu.VMEM((N,), jnp.float32),
            pltpu.VMEM((N,), jnp.float32),
            pltpu.VMEM((N,), jnp.float32),
            pltpu.VMEM((N,), jnp.float32),
            pltpu.VMEM((N,), jnp.float32),
            pltpu.VMEM((N,), jnp.float32),
            pltpu.VMEM((N,), jnp.float32),
            pltpu.VMEM((WPB * SPW,), jnp.float32),
            pltpu.VMEM((G * 2 * L,), jnp.float32),
            pltpu.VMEM((L,), jnp.float32),
            pltpu.VMEM_SHARED((NS * N,), jnp.float32),
        ],
    )
    return fn(in_flat, inb_flat, tgt_flat, tgtb_flat)


def _merge_body(tc_cm_ref, sc_cm_ref, rs_ref, out_ref):
    m = jnp.minimum(tc_cm_ref[:, 0, :], sc_cm_ref[...])
    out_ref[0, 0] = jnp.sum(m) + jnp.sum(rs_ref[...])


def _merge(tc_cm, sc_cm, sc_rs):
    out = pl.pallas_call(
        _merge_body,
        out_specs=pl.BlockSpec(memory_space=pltpu.SMEM),
        out_shape=jax.ShapeDtypeStruct((1, 1), jnp.float32),
    )(tc_cm, sc_cm, sc_rs)
    return out[0, 0]


def kernel(in_pc, target_pc):
    p1t = jnp.transpose(in_pc, (0, 2, 1))
    tc_loss, tc_cm = _tc_part(p1t[:, :TCR], target_pc)

    in_flat = in_pc.reshape(-1)
    tgt_flat = target_pc.reshape(-1)
    inb_flat = _bf16_round_f32(in_flat)
    tgtb_flat = _bf16_round_f32(tgt_flat)
    sc_out = _sc_part(in_flat, inb_flat, tgt_flat, tgtb_flat)

    sc_cm = sc_out[:B * N].reshape(B, N)
    sc_rs = sc_out[B * N:]
    rest = _merge(tc_cm, sc_cm, sc_rs)
    return (tc_loss[0, 0] + rest) / (2.0 * B * N)
